# baseline (speedup 1.0000x reference)
import jax
import jax.numpy as jnp
from jax import lax
from jax.experimental import pallas as pl
from jax.experimental.pallas import tpu as pltpu
from jax.experimental.pallas import tpu_sc as plsc

_NC = 2
_NS = 16
_NW = _NC * _NS
_CHUNK = 128


def _sc_gather_body(idx_hbm, table_hbm, out_hbm, idx_v, rows_v, sem):
    wid = lax.axis_index("s") * _NC + lax.axis_index("c")
    n_chunks = idx_v.shape[0]
    rows_per_w = n_chunks * _CHUNK
    pltpu.sync_copy(idx_hbm.at[wid], idx_v)
    copies = [
        pltpu.async_copy(
            table_hbm.at[idx_v.at[c]],
            rows_v.at[pl.ds(c * _CHUNK, _CHUNK)],
            sem,
        )
        for c in range(n_chunks)
    ]
    for cp in copies:
        cp.wait()
    pltpu.sync_copy(rows_v, out_hbm.at[pl.ds(wid * rows_per_w, rows_per_w)])


def _sc_gather(user_id, table):
    batch, dim = user_id.shape[0], table.shape[1]
    rows_per_w = batch // _NW
    n_chunks = rows_per_w // _CHUNK
    idx3 = user_id.reshape(_NW, n_chunks, _CHUNK)
    mesh = plsc.VectorSubcoreMesh(core_axis_name="c", subcore_axis_name="s")
    gather = pl.kernel(
        _sc_gather_body,
        out_type=jax.ShapeDtypeStruct((batch, dim), table.dtype),
        mesh=mesh,
        scratch_types=[
            pltpu.VMEM((n_chunks, _CHUNK), jnp.int32),
            pltpu.VMEM((rows_per_w, dim), table.dtype),
            pltpu.SemaphoreType.DMA,
        ],
        compiler_params=pltpu.CompilerParams(use_tc_tiling_on_sc=False),
    )
    return gather(idx3, table)


def _tc_body(u_ref, tm_ref, ts_ref, wu_ref, mc_ref, wt_ref, o_ref):
    u = u_ref[...].astype(jnp.float32)
    tm = tm_ref[0, 0, :]
    n_modes = mc_ref.shape[0]
    onehot = (
        tm[:, None] == lax.broadcasted_iota(jnp.int32, (1, n_modes), 1)
    ).astype(jnp.float32)
    acc = jnp.dot(u, wu_ref[...], preferred_element_type=jnp.float32)
    acc += jnp.dot(onehot, mc_ref[...], preferred_element_type=jnp.float32)
    acc += jnp.dot(ts_ref[...], wt_ref[...], preferred_element_type=jnp.float32)
    o_ref[...] = acc


def kernel(user_id, transport_mode, timestamp, user_table, mode_table,
           W_time, b_time, W_pref, b_pref):
    batch = user_id.shape[0]
    dim = user_table.shape[1]
    out_dim = W_pref.shape[0]
    n_modes = mode_table.shape[0]
    t_in = timestamp.shape[1]

    Wu = W_pref[:, :dim]
    Wm = W_pref[:, dim:2 * dim]
    Wt = W_pref[:, 2 * dim:3 * dim]
    mode_lut = mode_table @ Wm.T + b_pref + b_time @ Wt.T
    t_pad = 8
    W_ts = jnp.zeros((t_pad, out_dim), jnp.float32).at[:t_in].set((Wt @ W_time).T)
    ts_pad = jnp.zeros((batch, t_pad), jnp.float32).at[:, :t_in].set(timestamp)

    table_bf = user_table.astype(jnp.bfloat16)
    user_emb = _sc_gather(user_id.astype(jnp.int32), table_bf)

    blk = 2048
    n_blk = batch // blk
    tm3 = transport_mode.astype(jnp.int32).reshape(n_blk, 1, blk)

    return pl.pallas_call(
        _tc_body,
        grid=(n_blk,),
        in_specs=[
            pl.BlockSpec((blk, dim), lambda i: (i, 0)),
            pl.BlockSpec((1, 1, blk), lambda i: (i, 0, 0)),
            pl.BlockSpec((blk, t_pad), lambda i: (i, 0)),
            pl.BlockSpec((dim, out_dim), lambda i: (0, 0)),
            pl.BlockSpec((n_modes, out_dim), lambda i: (0, 0)),
            pl.BlockSpec((t_pad, out_dim), lambda i: (0, 0)),
        ],
        out_specs=pl.BlockSpec((blk, out_dim), lambda i: (i, 0)),
        out_shape=jax.ShapeDtypeStruct((batch, out_dim), jnp.float32),
    )(user_emb, tm3, ts_pad, Wu.T, mode_lut, W_ts)


# speedup vs baseline: 1.8178x; 1.8178x over previous
import jax
import jax.numpy as jnp
from jax import lax
from jax.experimental import pallas as pl
from jax.experimental.pallas import tpu as pltpu
from jax.experimental.pallas import tpu_sc as plsc

_NC = 2
_NS = 16
_NW = _NC * _NS
_CHUNK = 128
_LANES = 128
_TP_W = 4096


_TP_REM = 512


def _tp_body(tbl_ref, tail_ref, o_ref, buf0, buf1, sems):
    c = pl.program_id(0)
    n = pl.num_programs(0)
    dim, w = buf0.shape

    def start(chunk, buf, slot):
        @pl.when(chunk < n - 1)
        def _():
            pltpu.make_async_copy(
                tbl_ref.at[:, pl.ds(chunk * w, w)], buf, sems.at[slot]
            ).start()
        @pl.when(chunk == n - 1)
        def _():
            pltpu.make_async_copy(
                tbl_ref.at[:, pl.ds(chunk * w, _TP_REM)],
                buf.at[:, pl.ds(0, _TP_REM)], sems.at[slot],
            ).start()

    def wait(chunk, buf, slot):
        @pl.when(chunk < n - 1)
        def _():
            pltpu.make_async_copy(
                tbl_ref.at[:, pl.ds(chunk * w, w)], buf, sems.at[slot]
            ).wait()
        @pl.when(chunk == n - 1)
        def _():
            pltpu.make_async_copy(
                tbl_ref.at[:, pl.ds(chunk * w, _TP_REM)],
                buf.at[:, pl.ds(0, _TP_REM)], sems.at[slot],
            ).wait()

    def compute(buf):
        @pl.when(c < n - 1)
        def _():
            x = buf[...]
            xT = x.T
            g = w // 4
            for k in range(4):
                o_ref[:, pl.ds(dim * k, dim)] = xT[g * k:g * (k + 1), :]
        @pl.when(c == n - 1)
        def _():
            x = buf[:, pl.ds(0, _TP_REM)]
            xT = x.T
            g = _TP_REM // 4
            t = tail_ref[...]
            gt = t.shape[0] // 4
            for k in range(4):
                o_ref[pl.ds(0, g), pl.ds(dim * k, dim)] = xT[g * k:g * (k + 1), :]
                o_ref[pl.ds(g, gt), pl.ds(dim * k, dim)] = t[gt * k:gt * (k + 1), :]

    @pl.when(c == 0)
    def _():
        start(0, buf0, 0)

    @pl.when((c + 1 < n) & ((c + 1) % 2 == 0))
    def _():
        start(c + 1, buf0, 0)

    @pl.when((c + 1 < n) & ((c + 1) % 2 == 1))
    def _():
        start(c + 1, buf1, 1)

    @pl.when(c % 2 == 0)
    def _():
        wait(c, buf0, 0)
        compute(buf0)

    @pl.when(c % 2 == 1)
    def _():
        wait(c, buf1, 1)
        compute(buf1)


def _pack_table(tableT, tail):
    dim, n_users = tableT.shape
    grid = (n_users + _TP_W - 1) // _TP_W
    return pl.pallas_call(
        _tp_body,
        grid=(grid,),
        in_specs=[
            pl.BlockSpec(memory_space=pl.ANY),
            pl.BlockSpec(tail.shape, lambda i: (0, 0)),
        ],
        out_specs=pl.BlockSpec((_TP_W // 4, _LANES), lambda i: (i, 0)),
        out_shape=jax.ShapeDtypeStruct((n_users // 4, _LANES), jnp.float32),
        scratch_shapes=[
            pltpu.VMEM((dim, _TP_W), jnp.float32),
            pltpu.VMEM((dim, _TP_W), jnp.float32),
            pltpu.SemaphoreType.DMA((2,)),
        ],
    )(tableT, tail)


def _sc_gather_body(idx_hbm, table_hbm, out_hbm, idx_v, rows_v, sem):
    wid = lax.axis_index("s") * _NC + lax.axis_index("c")
    n_chunks = idx_v.shape[0]
    rows_per_w = n_chunks * _CHUNK
    pltpu.sync_copy(idx_hbm.at[wid], idx_v)
    copies = [
        pltpu.async_copy(
            table_hbm.at[idx_v.at[c]],
            rows_v.at[pl.ds(c * _CHUNK, _CHUNK)],
            sem,
        )
        for c in range(n_chunks)
    ]
    for cp in copies:
        cp.wait()
    pltpu.sync_copy(rows_v, out_hbm.at[pl.ds(wid * rows_per_w, rows_per_w)])


def _sc_gather_lines(line_idx, table2):
    batch = line_idx.shape[0]
    rows_per_w = batch // _NW
    n_chunks = rows_per_w // _CHUNK
    idx3 = line_idx.reshape(_NW, n_chunks, _CHUNK)
    mesh = plsc.VectorSubcoreMesh(core_axis_name="c", subcore_axis_name="s")
    gather = pl.kernel(
        _sc_gather_body,
        out_type=jax.ShapeDtypeStruct((batch, _LANES), jnp.float32),
        mesh=mesh,
        scratch_types=[
            pltpu.VMEM((n_chunks, _CHUNK), jnp.int32),
            pltpu.VMEM((rows_per_w, _LANES), jnp.float32),
            pltpu.SemaphoreType.DMA,
        ],
    )
    return gather(idx3, table2)


def _tc_body(ul_ref, sub_ref, tm_ref, ts_ref, wu_ref, mc_ref, wt_ref, o_ref):
    dim = wu_ref.shape[0]
    lines = ul_ref[...]
    sub = sub_ref[0, 0, :][:, None]
    u = lines[:, 3 * dim:4 * dim]
    for k in (2, 1, 0):
        u = jnp.where(sub == k, lines[:, k * dim:(k + 1) * dim], u)
    tm = tm_ref[0, 0, :]
    n_modes = mc_ref.shape[0]
    onehot = (
        tm[:, None] == lax.broadcasted_iota(jnp.int32, (1, n_modes), 1)
    ).astype(jnp.float32)
    acc = jnp.dot(u, wu_ref[...], preferred_element_type=jnp.float32)
    acc += jnp.dot(onehot, mc_ref[...], preferred_element_type=jnp.float32)
    acc += jnp.dot(ts_ref[...], wt_ref[...], preferred_element_type=jnp.float32)
    o_ref[...] = acc


def kernel(user_id, transport_mode, timestamp, user_table, mode_table,
           W_time, b_time, W_pref, b_pref):
    batch = user_id.shape[0]
    dim = user_table.shape[1]
    out_dim = W_pref.shape[0]
    n_modes = mode_table.shape[0]
    t_in = timestamp.shape[1]
    per_line = _LANES // dim

    Wu = W_pref[:, :dim]
    Wm = W_pref[:, dim:2 * dim]
    Wt = W_pref[:, 2 * dim:3 * dim]
    mode_lut = mode_table @ Wm.T + b_pref + b_time @ Wt.T
    t_pad = 8
    W_ts = jnp.zeros((t_pad, out_dim), jnp.float32).at[:t_in].set((Wt @ W_time).T)
    ts_pad = jnp.zeros((batch, t_pad), jnp.float32).at[:, :t_in].set(timestamp)

    uid = user_id.astype(jnp.int32)
    n_users = user_table.shape[0]
    n_chunks = n_users // _TP_W
    main_lim = n_chunks * _TP_W
    dma_lim = main_lim + _TP_REM
    table2 = _pack_table(user_table.T, user_table[dma_lim:, :])

    g_main, g_rem = _TP_W // 4, _TP_REM // 4
    g_tail = (n_users - dma_lim) // 4
    c = uid // _TP_W
    p = uid % _TP_W
    q1 = uid - main_lim
    q2 = uid - dma_lim
    line_idx = jnp.where(
        uid < main_lim, g_main * c + p % g_main,
        jnp.where(uid < dma_lim, n_chunks * g_main + q1 % g_rem,
                  n_chunks * g_main + g_rem + q2 % g_tail))
    sub = jnp.where(
        uid < main_lim, p // g_main,
        jnp.where(uid < dma_lim, q1 // g_rem, q2 // g_tail))
    user_lines = _sc_gather_lines(line_idx, table2)

    blk = 2048
    n_blk = batch // blk
    sub3 = sub.reshape(n_blk, 1, blk)
    tm3 = transport_mode.astype(jnp.int32).reshape(n_blk, 1, blk)

    return pl.pallas_call(
        _tc_body,
        grid=(n_blk,),
        in_specs=[
            pl.BlockSpec((blk, _LANES), lambda i: (i, 0)),
            pl.BlockSpec((1, 1, blk), lambda i: (i, 0, 0)),
            pl.BlockSpec((1, 1, blk), lambda i: (i, 0, 0)),
            pl.BlockSpec((blk, t_pad), lambda i: (i, 0)),
            pl.BlockSpec((dim, out_dim), lambda i: (0, 0)),
            pl.BlockSpec((n_modes, out_dim), lambda i: (0, 0)),
            pl.BlockSpec((t_pad, out_dim), lambda i: (0, 0)),
        ],
        out_specs=pl.BlockSpec((blk, out_dim), lambda i: (i, 0)),
        out_shape=jax.ShapeDtypeStruct((batch, out_dim), jnp.float32),
    )(user_lines, sub3, tm3, ts_pad, Wu.T, mode_lut, W_ts)


# speedup vs baseline: 2.0259x; 1.1144x over previous
import jax
import jax.numpy as jnp
from jax import lax
from jax.experimental import pallas as pl
from jax.experimental.pallas import tpu as pltpu
from jax.experimental.pallas import tpu_sc as plsc

_NC = 2
_NS = 16
_NW = _NC * _NS
_CHUNK = 128
_LANES = 128
_TP_W = 4096


_TP_REM = 512


def _tp_body(tbl_ref, tail_ref, sel_ref, o_ref, buf0, buf1, sems):
    c = pl.program_id(0)
    n = pl.num_programs(0)
    dim, w = buf0.shape

    def start(chunk, buf, slot):
        @pl.when(chunk < n - 1)
        def _():
            pltpu.make_async_copy(
                tbl_ref.at[:, pl.ds(chunk * w, w)], buf, sems.at[slot]
            ).start()
        @pl.when(chunk == n - 1)
        def _():
            pltpu.make_async_copy(
                tbl_ref.at[:, pl.ds(chunk * w, _TP_REM)],
                buf.at[:, pl.ds(0, _TP_REM)], sems.at[slot],
            ).start()

    def wait(chunk, buf, slot):
        @pl.when(chunk < n - 1)
        def _():
            pltpu.make_async_copy(
                tbl_ref.at[:, pl.ds(chunk * w, w)], buf, sems.at[slot]
            ).wait()
        @pl.when(chunk == n - 1)
        def _():
            pltpu.make_async_copy(
                tbl_ref.at[:, pl.ds(chunk * w, _TP_REM)],
                buf.at[:, pl.ds(0, _TP_REM)], sems.at[slot],
            ).wait()

    def xpose_mxu(x):
        g = x.shape[1] // 4
        acc = None
        for k in range(4):
            xk = x[:, g * k:g * (k + 1)]
            sk = sel_ref[:, _LANES * k:_LANES * (k + 1)]
            t = lax.dot_general(
                xk, sk, (((0,), (0,)), ((), ())),
                preferred_element_type=jnp.float32)
            acc = t if acc is None else acc + t
        return acc

    def compute(buf):
        @pl.when(c < n - 1)
        def _():
            o_ref[...] = xpose_mxu(buf[...])
        @pl.when(c == n - 1)
        def _():
            g = _TP_REM // 4
            o_ref[pl.ds(0, g), :] = xpose_mxu(buf[:, pl.ds(0, _TP_REM)])
            t = tail_ref[...]
            gt = t.shape[0] // 4
            for k in range(4):
                o_ref[pl.ds(g, gt), pl.ds(dim * k, dim)] = t[gt * k:gt * (k + 1), :]

    @pl.when(c == 0)
    def _():
        start(0, buf0, 0)

    @pl.when((c + 1 < n) & ((c + 1) % 2 == 0))
    def _():
        start(c + 1, buf0, 0)

    @pl.when((c + 1 < n) & ((c + 1) % 2 == 1))
    def _():
        start(c + 1, buf1, 1)

    @pl.when(c % 2 == 0)
    def _():
        wait(c, buf0, 0)
        compute(buf0)

    @pl.when(c % 2 == 1)
    def _():
        wait(c, buf1, 1)
        compute(buf1)


def _pack_table(tableT, tail):
    dim, n_users = tableT.shape
    grid = (n_users + _TP_W - 1) // _TP_W
    lane = jnp.arange(4 * _LANES, dtype=jnp.int32)
    row = jnp.arange(dim, dtype=jnp.int32)[:, None]
    sel = (lane[None, :] % _LANES == dim * (lane[None, :] // _LANES) + row)
    sel = sel.astype(jnp.float32)
    return pl.pallas_call(
        _tp_body,
        grid=(grid,),
        in_specs=[
            pl.BlockSpec(memory_space=pl.ANY),
            pl.BlockSpec(tail.shape, lambda i: (0, 0)),
            pl.BlockSpec(sel.shape, lambda i: (0, 0)),
        ],
        out_specs=pl.BlockSpec((_TP_W // 4, _LANES), lambda i: (i, 0)),
        out_shape=jax.ShapeDtypeStruct((n_users // 4, _LANES), jnp.float32),
        scratch_shapes=[
            pltpu.VMEM((dim, _TP_W), jnp.float32),
            pltpu.VMEM((dim, _TP_W), jnp.float32),
            pltpu.SemaphoreType.DMA((2,)),
        ],
    )(tableT, tail, sel)


def _sc_gather_body(idx_hbm, table_hbm, out_hbm, idx_v, rows_v, sem):
    wid = lax.axis_index("s") * _NC + lax.axis_index("c")
    n_chunks = idx_v.shape[0]
    rows_per_w = n_chunks * _CHUNK
    pltpu.sync_copy(idx_hbm.at[wid], idx_v)
    copies = [
        pltpu.async_copy(
            table_hbm.at[idx_v.at[c]],
            rows_v.at[pl.ds(c * _CHUNK, _CHUNK)],
            sem,
        )
        for c in range(n_chunks)
    ]
    for cp in copies:
        cp.wait()
    pltpu.sync_copy(rows_v, out_hbm.at[pl.ds(wid * rows_per_w, rows_per_w)])


def _sc_gather_lines(line_idx, table2):
    batch = line_idx.shape[0]
    rows_per_w = batch // _NW
    n_chunks = rows_per_w // _CHUNK
    idx3 = line_idx.reshape(_NW, n_chunks, _CHUNK)
    mesh = plsc.VectorSubcoreMesh(core_axis_name="c", subcore_axis_name="s")
    gather = pl.kernel(
        _sc_gather_body,
        out_type=jax.ShapeDtypeStruct((batch, _LANES), jnp.float32),
        mesh=mesh,
        scratch_types=[
            pltpu.VMEM((n_chunks, _CHUNK), jnp.int32),
            pltpu.VMEM((rows_per_w, _LANES), jnp.float32),
            pltpu.SemaphoreType.DMA,
        ],
    )
    return gather(idx3, table2)


def _tc_body(ul_ref, sub_ref, tm_ref, ts_ref, wu_ref, mc_ref, wt_ref, o_ref):
    dim = wu_ref.shape[0]
    lines = ul_ref[...]
    sub = sub_ref[0, 0, :][:, None]
    u = lines[:, 3 * dim:4 * dim]
    for k in (2, 1, 0):
        u = jnp.where(sub == k, lines[:, k * dim:(k + 1) * dim], u)
    tm = tm_ref[0, 0, :]
    n_modes = mc_ref.shape[0]
    onehot = (
        tm[:, None] == lax.broadcasted_iota(jnp.int32, (1, n_modes), 1)
    ).astype(jnp.float32)
    acc = jnp.dot(u, wu_ref[...], preferred_element_type=jnp.float32)
    acc += jnp.dot(onehot, mc_ref[...], preferred_element_type=jnp.float32)
    acc += jnp.dot(ts_ref[...], wt_ref[...], preferred_element_type=jnp.float32)
    o_ref[...] = acc


def kernel(user_id, transport_mode, timestamp, user_table, mode_table,
           W_time, b_time, W_pref, b_pref):
    batch = user_id.shape[0]
    dim = user_table.shape[1]
    out_dim = W_pref.shape[0]
    n_modes = mode_table.shape[0]
    t_in = timestamp.shape[1]
    per_line = _LANES // dim

    Wu = W_pref[:, :dim]
    Wm = W_pref[:, dim:2 * dim]
    Wt = W_pref[:, 2 * dim:3 * dim]
    mode_lut = mode_table @ Wm.T + b_pref + b_time @ Wt.T
    t_pad = 8
    W_ts = jnp.zeros((t_pad, out_dim), jnp.float32).at[:t_in].set((Wt @ W_time).T)
    ts_pad = jnp.zeros((batch, t_pad), jnp.float32).at[:, :t_in].set(timestamp)

    uid = user_id.astype(jnp.int32)
    n_users = user_table.shape[0]
    n_chunks = n_users // _TP_W
    main_lim = n_chunks * _TP_W
    dma_lim = main_lim + _TP_REM
    table2 = _pack_table(user_table.T, user_table[dma_lim:, :])

    g_main, g_rem = _TP_W // 4, _TP_REM // 4
    g_tail = (n_users - dma_lim) // 4
    c = uid // _TP_W
    p = uid % _TP_W
    q1 = uid - main_lim
    q2 = uid - dma_lim
    line_idx = jnp.where(
        uid < main_lim, g_main * c + p % g_main,
        jnp.where(uid < dma_lim, n_chunks * g_main + q1 % g_rem,
                  n_chunks * g_main + g_rem + q2 % g_tail))
    sub = jnp.where(
        uid < main_lim, p // g_main,
        jnp.where(uid < dma_lim, q1 // g_rem, q2 // g_tail))
    user_lines = _sc_gather_lines(line_idx, table2)

    blk = 2048
    n_blk = batch // blk
    sub3 = sub.reshape(n_blk, 1, blk)
    tm3 = transport_mode.astype(jnp.int32).reshape(n_blk, 1, blk)

    return pl.pallas_call(
        _tc_body,
        grid=(n_blk,),
        in_specs=[
            pl.BlockSpec((blk, _LANES), lambda i: (i, 0)),
            pl.BlockSpec((1, 1, blk), lambda i: (i, 0, 0)),
            pl.BlockSpec((1, 1, blk), lambda i: (i, 0, 0)),
            pl.BlockSpec((blk, t_pad), lambda i: (i, 0)),
            pl.BlockSpec((dim, out_dim), lambda i: (0, 0)),
            pl.BlockSpec((n_modes, out_dim), lambda i: (0, 0)),
            pl.BlockSpec((t_pad, out_dim), lambda i: (0, 0)),
        ],
        out_specs=pl.BlockSpec((blk, out_dim), lambda i: (i, 0)),
        out_shape=jax.ShapeDtypeStruct((batch, out_dim), jnp.float32),
    )(user_lines, sub3, tm3, ts_pad, Wu.T, mode_lut, W_ts)


# speedup vs baseline: 2.0391x; 1.0065x over previous
import jax
import jax.numpy as jnp
from jax import lax
from jax.experimental import pallas as pl
from jax.experimental.pallas import tpu as pltpu
from jax.experimental.pallas import tpu_sc as plsc

_NC = 2
_NS = 16
_NW = _NC * _NS
_CHUNK = 128
_LANES = 128
_TP_W = 4096


_TP_REM = 512


def _tp_body(x_ref, tail_ref, sel_ref, o_ref):
    c = pl.program_id(0)
    n = pl.num_programs(0)
    dim = x_ref.shape[0]

    def xpose_mxu(x):
        g = x.shape[1] // 4
        acc = None
        for k in range(4):
            xk = x[:, g * k:g * (k + 1)]
            sk = sel_ref[:, _LANES * k:_LANES * (k + 1)]
            t = lax.dot_general(
                xk, sk, (((0,), (0,)), ((), ())),
                preferred_element_type=jnp.float32)
            acc = t if acc is None else acc + t
        return acc

    @pl.when(c < n - 1)
    def _():
        o_ref[...] = xpose_mxu(x_ref[...])

    @pl.when(c == n - 1)
    def _():
        g = _TP_REM // 4
        o_ref[pl.ds(0, g), :] = xpose_mxu(x_ref[:, pl.ds(0, _TP_REM)])
        t = tail_ref[...]
        gt = t.shape[0] // 4
        for k in range(4):
            o_ref[pl.ds(g, gt), pl.ds(dim * k, dim)] = t[gt * k:gt * (k + 1), :]


def _pack_table(tableT, tail):
    dim, n_users = tableT.shape
    grid = (n_users + _TP_W - 1) // _TP_W
    lane = jnp.arange(4 * _LANES, dtype=jnp.int32)
    row = jnp.arange(dim, dtype=jnp.int32)[:, None]
    sel = (lane[None, :] % _LANES == dim * (lane[None, :] // _LANES) + row)
    sel = sel.astype(jnp.float32)
    return pl.pallas_call(
        _tp_body,
        grid=(grid,),
        in_specs=[
            pl.BlockSpec((dim, _TP_W), lambda i: (0, i)),
            pl.BlockSpec(tail.shape, lambda i: (0, 0)),
            pl.BlockSpec(sel.shape, lambda i: (0, 0)),
        ],
        out_specs=pl.BlockSpec((_TP_W // 4, _LANES), lambda i: (i, 0)),
        out_shape=jax.ShapeDtypeStruct((n_users // 4, _LANES), jnp.float32),
        compiler_params=pltpu.CompilerParams(
            dimension_semantics=("parallel",)),
    )(tableT, tail, sel)


def _sc_gather_body(idx_hbm, table_hbm, out_hbm, idx_v, rows_v, sem):
    wid = lax.axis_index("s") * _NC + lax.axis_index("c")
    n_chunks = idx_v.shape[0]
    rows_per_w = n_chunks * _CHUNK
    pltpu.sync_copy(idx_hbm.at[wid], idx_v)
    copies = [
        pltpu.async_copy(
            table_hbm.at[idx_v.at[c]],
            rows_v.at[pl.ds(c * _CHUNK, _CHUNK)],
            sem,
        )
        for c in range(n_chunks)
    ]
    for cp in copies:
        cp.wait()
    pltpu.sync_copy(rows_v, out_hbm.at[pl.ds(wid * rows_per_w, rows_per_w)])


def _sc_gather_lines(line_idx, table2):
    batch = line_idx.shape[0]
    rows_per_w = batch // _NW
    n_chunks = rows_per_w // _CHUNK
    idx3 = line_idx.reshape(_NW, n_chunks, _CHUNK)
    mesh = plsc.VectorSubcoreMesh(core_axis_name="c", subcore_axis_name="s")
    gather = pl.kernel(
        _sc_gather_body,
        out_type=jax.ShapeDtypeStruct((batch, _LANES), jnp.float32),
        mesh=mesh,
        scratch_types=[
            pltpu.VMEM((n_chunks, _CHUNK), jnp.int32),
            pltpu.VMEM((rows_per_w, _LANES), jnp.float32),
            pltpu.SemaphoreType.DMA,
        ],
    )
    return gather(idx3, table2)


def _tc_body(ul_ref, sub_ref, tm_ref, ts_ref, wu_ref, mc_ref, wt_ref, o_ref):
    dim = wu_ref.shape[0]
    lines = ul_ref[...]
    sub = sub_ref[0, 0, :][:, None]
    u = lines[:, 3 * dim:4 * dim]
    for k in (2, 1, 0):
        u = jnp.where(sub == k, lines[:, k * dim:(k + 1) * dim], u)
    tm = tm_ref[0, 0, :]
    n_modes = mc_ref.shape[0]
    onehot = (
        tm[:, None] == lax.broadcasted_iota(jnp.int32, (1, n_modes), 1)
    ).astype(jnp.float32)
    acc = jnp.dot(u, wu_ref[...], preferred_element_type=jnp.float32)
    acc += jnp.dot(onehot, mc_ref[...], preferred_element_type=jnp.float32)
    acc += jnp.dot(ts_ref[...], wt_ref[...], preferred_element_type=jnp.float32)
    o_ref[...] = acc


def kernel(user_id, transport_mode, timestamp, user_table, mode_table,
           W_time, b_time, W_pref, b_pref):
    batch = user_id.shape[0]
    dim = user_table.shape[1]
    out_dim = W_pref.shape[0]
    n_modes = mode_table.shape[0]
    t_in = timestamp.shape[1]
    per_line = _LANES // dim

    Wu = W_pref[:, :dim]
    Wm = W_pref[:, dim:2 * dim]
    Wt = W_pref[:, 2 * dim:3 * dim]
    mode_lut = mode_table @ Wm.T + b_pref + b_time @ Wt.T
    t_pad = 8
    W_ts = jnp.zeros((t_pad, out_dim), jnp.float32).at[:t_in].set((Wt @ W_time).T)
    ts_pad = jnp.zeros((batch, t_pad), jnp.float32).at[:, :t_in].set(timestamp)

    uid = user_id.astype(jnp.int32)
    n_users = user_table.shape[0]
    n_chunks = n_users // _TP_W
    main_lim = n_chunks * _TP_W
    dma_lim = main_lim + _TP_REM
    table2 = _pack_table(user_table.T, user_table[dma_lim:, :])

    g_main, g_rem = _TP_W // 4, _TP_REM // 4
    g_tail = (n_users - dma_lim) // 4
    c = uid // _TP_W
    p = uid % _TP_W
    q1 = uid - main_lim
    q2 = uid - dma_lim
    line_idx = jnp.where(
        uid < main_lim, g_main * c + p % g_main,
        jnp.where(uid < dma_lim, n_chunks * g_main + q1 % g_rem,
                  n_chunks * g_main + g_rem + q2 % g_tail))
    sub = jnp.where(
        uid < main_lim, p // g_main,
        jnp.where(uid < dma_lim, q1 // g_rem, q2 // g_tail))
    user_lines = _sc_gather_lines(line_idx, table2)

    blk = 2048
    n_blk = batch // blk
    sub3 = sub.reshape(n_blk, 1, blk)
    tm3 = transport_mode.astype(jnp.int32).reshape(n_blk, 1, blk)

    return pl.pallas_call(
        _tc_body,
        grid=(n_blk,),
        in_specs=[
            pl.BlockSpec((blk, _LANES), lambda i: (i, 0)),
            pl.BlockSpec((1, 1, blk), lambda i: (i, 0, 0)),
            pl.BlockSpec((1, 1, blk), lambda i: (i, 0, 0)),
            pl.BlockSpec((blk, t_pad), lambda i: (i, 0)),
            pl.BlockSpec((dim, out_dim), lambda i: (0, 0)),
            pl.BlockSpec((n_modes, out_dim), lambda i: (0, 0)),
            pl.BlockSpec((t_pad, out_dim), lambda i: (0, 0)),
        ],
        out_specs=pl.BlockSpec((blk, out_dim), lambda i: (i, 0)),
        out_shape=jax.ShapeDtypeStruct((batch, out_dim), jnp.float32),
    )(user_lines, sub3, tm3, ts_pad, Wu.T, mode_lut, W_ts)
